# Initial kernel scaffold; baseline (speedup 1.0000x reference)
import jax
import jax.numpy as jnp
from jax.experimental import pallas as pl


def kernel(x, conv_w, bn_gamma, bn_beta, lin_w):
    raise NotImplementedError("write your pallas kernel here")


# speedup vs baseline: 838.4231x; 838.4231x over previous
import jax
import jax.numpy as jnp
from jax.experimental import pallas as pl

_B = 4
_N = 4096
_K = 20
_NC = 40


def _roll(v, s):
    if s % _N == 0:
        return v
    return jnp.roll(v, s, axis=1)


def _body(x_ref, w_ref, g_ref, bt_ref, lw_ref, out_ref):
    f32 = jnp.float32
    lane = jax.lax.broadcasted_iota(jnp.int32, (_B, _N), 1)

    v = x_ref[...]
    k = 2
    while k <= _N:
        j = k // 2
        while j >= 1:
            partner = jnp.where((lane & j) == 0,
                                _roll(v, -j),
                                _roll(v, j))
            keep_min = ((lane & k) == 0) == ((lane & j) == 0)
            v = jnp.where(keep_min, jnp.minimum(v, partner),
                          jnp.maximum(v, partner))
            j //= 2
        k *= 2

    c1 = v
    c2 = v * v
    t = 1
    while t < _N:
        c1 = c1 + jnp.where(lane >= t, _roll(c1, t), 0.0)
        c2 = c2 + jnp.where(lane >= t, _roll(c2, t), 0.0)
        t *= 2

    big = f32(1e30)
    best_m = jnp.full((_B, _N), big, f32)
    bvmin = jnp.zeros((_B, _N), f32)
    bvmax = jnp.zeros((_B, _N), f32)
    bsv = jnp.zeros((_B, _N), f32)
    bsv2 = jnp.zeros((_B, _N), f32)
    for o in range(_K):
        h = _K - 1 - o
        s_lo = _roll(v, o)
        s_hi = _roll(v, -h)
        cm = jnp.maximum(v - s_lo, s_hi - v)
        valid = (lane >= o) & (lane <= (_N - _K) + o)
        cm = jnp.where(valid, cm, big)
        upd = cm < best_m
        best_m = jnp.where(upd, cm, best_m)
        bvmin = jnp.where(upd, s_lo, bvmin)
        bvmax = jnp.where(upd, s_hi, bvmax)
        c_lo = _roll(c1, o)
        c_hi = _roll(c1, -h)
        q_lo = _roll(c2, o)
        q_hi = _roll(c2, -h)
        bsv = jnp.where(upd, c_hi - c_lo + s_lo, bsv)
        bsv2 = jnp.where(upd, q_hi - q_lo + s_lo * s_lo, bsv2)

    s1 = jnp.sum(bsv)
    s2 = jnp.sum(bsv2)
    s3 = jnp.sum(bsv * v)
    sx = jnp.sum(v)
    sx2 = jnp.sum(v * v)

    aw = w_ref[:, 0:1]
    cw = w_ref[:, 1:2] - aw
    m = f32(_B * _N * _K)
    mean = (aw * s1 + cw * (f32(_K) * sx)) / m
    ey2 = (aw * aw * s2 + 2.0 * aw * cw * s3 + cw * cw * (f32(_K) * sx2)) / m
    var = ey2 - mean * mean
    inv = jax.lax.rsqrt(var + 1e-5)
    g = g_ref[...]
    amp = g * aw * inv
    cc = g * cw * inv
    d = bt_ref[...] - g * mean * inv

    x1s = []
    x2s = []
    for b in range(_B):
        sb = v[b:b + 1, :]
        vx = jnp.where(amp > 0, bvmax[b:b + 1, :], bvmin[b:b + 1, :])
        z = jnp.maximum(amp * vx + cc * sb + d, 0.0)
        x1s.append(jnp.max(z, axis=1, keepdims=True))
        x2s.append(jnp.sum(z, axis=1, keepdims=True) * f32(1.0 / _N))
    xh = jnp.concatenate([jnp.concatenate(x1s, axis=1),
                          jnp.concatenate(x2s, axis=1)], axis=0)
    out_ref[...] = jnp.dot(lw_ref[...], xh,
                           preferred_element_type=jnp.float32)


def kernel(x, conv_w, bn_gamma, bn_beta, lin_w):
    out = pl.pallas_call(
        _body,
        out_shape=jax.ShapeDtypeStruct((_NC, _B), jnp.float32),
    )(x, conv_w, bn_gamma.reshape(64, 1), bn_beta.reshape(64, 1), lin_w)
    return out.T


# speedup vs baseline: 922.6510x; 1.1005x over previous
import jax
import jax.numpy as jnp
from jax.experimental import pallas as pl

_B = 4
_N = 4096
_K = 20
_NC = 40


def _roll(v, s):
    if s % _N == 0:
        return v
    return jnp.roll(v, s, axis=1)


def _body(x_ref, w_ref, g_ref, bt_ref, lw_ref, out_ref):
    f32 = jnp.float32
    lane = jax.lax.broadcasted_iota(jnp.int32, (_B, _N), 1)

    v = x_ref[...]
    k = 2
    while k <= _N:
        j = k // 2
        while j >= 1:
            partner = jnp.where((lane & j) == 0,
                                _roll(v, -j),
                                _roll(v, j))
            keep_min = ((lane & k) == 0) == ((lane & j) == 0)
            v = jnp.where(keep_min, jnp.minimum(v, partner),
                          jnp.maximum(v, partner))
            j //= 2
        k *= 2

    vv = v * v
    a2 = v + _roll(v, -1)
    b2 = vv + _roll(vv, -1)
    a4 = a2 + _roll(a2, -2)
    b4 = b2 + _roll(b2, -2)
    a8 = a4 + _roll(a4, -4)
    b8 = b4 + _roll(b4, -4)
    a16 = a8 + _roll(a8, -8)
    b16 = b8 + _roll(b8, -8)
    w1 = a16 + _roll(a4, -16)
    w2 = b16 + _roll(b4, -16)

    big = f32(1e30)
    best_m = jnp.full((_B, _N), big, f32)
    bvmin = jnp.zeros((_B, _N), f32)
    bvmax = jnp.zeros((_B, _N), f32)
    bsv = jnp.zeros((_B, _N), f32)
    bsv2 = jnp.zeros((_B, _N), f32)
    for o in range(_K):
        h = _K - 1 - o
        s_lo = _roll(v, o)
        s_hi = _roll(v, -h)
        cm = jnp.maximum(v - s_lo, s_hi - v)
        valid = (lane >= o) & (lane <= (_N - _K) + o)
        cm = jnp.where(valid, cm, big)
        upd = cm < best_m
        best_m = jnp.where(upd, cm, best_m)
        bvmin = jnp.where(upd, s_lo, bvmin)
        bvmax = jnp.where(upd, s_hi, bvmax)
        bsv = jnp.where(upd, _roll(w1, o), bsv)
        bsv2 = jnp.where(upd, _roll(w2, o), bsv2)

    s1 = jnp.sum(bsv)
    s2 = jnp.sum(bsv2)
    s3 = jnp.sum(bsv * v)
    sx = jnp.sum(v)
    sx2 = jnp.sum(v * v)

    aw = w_ref[:, 0:1]
    cw = w_ref[:, 1:2] - aw
    m = f32(_B * _N * _K)
    mean = (aw * s1 + cw * (f32(_K) * sx)) / m
    ey2 = (aw * aw * s2 + 2.0 * aw * cw * s3 + cw * cw * (f32(_K) * sx2)) / m
    var = ey2 - mean * mean
    inv = jax.lax.rsqrt(var + 1e-5)
    g = g_ref[...]
    amp = g * aw * inv
    cc = g * cw * inv
    d = bt_ref[...] - g * mean * inv

    x1s = []
    x2s = []
    for b in range(_B):
        sb = v[b:b + 1, :]
        vx = jnp.where(amp > 0, bvmax[b:b + 1, :], bvmin[b:b + 1, :])
        z = jnp.maximum(amp * vx + cc * sb + d, 0.0)
        x1s.append(jnp.max(z, axis=1, keepdims=True))
        x2s.append(jnp.sum(z, axis=1, keepdims=True) * f32(1.0 / _N))
    xh = jnp.concatenate([jnp.concatenate(x1s, axis=1),
                          jnp.concatenate(x2s, axis=1)], axis=0)
    out_ref[...] = jnp.dot(lw_ref[...], xh,
                           preferred_element_type=jnp.float32)


def kernel(x, conv_w, bn_gamma, bn_beta, lin_w):
    out = pl.pallas_call(
        _body,
        out_shape=jax.ShapeDtypeStruct((_NC, _B), jnp.float32),
    )(x, conv_w, bn_gamma.reshape(64, 1), bn_beta.reshape(64, 1), lin_w)
    return out.T


# speedup vs baseline: 1088.3432x; 1.1796x over previous
import jax
import jax.numpy as jnp
from jax.experimental import pallas as pl

_B = 4
_N = 4096
_K = 20
_NC = 40


def _roll(v, s):
    if s % _N == 0:
        return v
    return jnp.roll(v, s, axis=1)


def _body(x_ref, w_ref, g_ref, bt_ref, lw_ref, out_ref):
    f32 = jnp.float32
    lane = jax.lax.broadcasted_iota(jnp.int32, (_B, _N), 1)

    v = x_ref[...]
    k = 2
    while k <= _N:
        j = k // 2
        while j >= 1:
            partner = jnp.where((lane & j) == 0,
                                _roll(v, -j),
                                _roll(v, j))
            keep_min = ((lane & k) == 0) == ((lane & j) == 0)
            v = jnp.where(keep_min, jnp.minimum(v, partner),
                          jnp.maximum(v, partner))
            j //= 2
        k *= 2

    vv = v * v
    a2 = v + _roll(v, -1)
    b2 = vv + _roll(vv, -1)
    a4 = a2 + _roll(a2, -2)
    b4 = b2 + _roll(b2, -2)
    a8 = a4 + _roll(a4, -4)
    b8 = b4 + _roll(b4, -4)
    a16 = a8 + _roll(a8, -8)
    b16 = b8 + _roll(b8, -8)
    w1 = a16 + _roll(a4, -16)
    w2 = b16 + _roll(b4, -16)

    big = f32(1e30)
    best_m = jnp.full((_B, _N), big, f32)
    bvmin = jnp.zeros((_B, _N), f32)
    bvmax = jnp.zeros((_B, _N), f32)
    bsv = jnp.zeros((_B, _N), f32)
    bsv2 = jnp.zeros((_B, _N), f32)
    for o in range(_K):
        h = _K - 1 - o
        s_lo = _roll(v, o)
        s_hi = _roll(v, -h)
        cm = jnp.maximum(v - s_lo, s_hi - v)
        valid = (lane >= o) & (lane <= (_N - _K) + o)
        cm = jnp.where(valid, cm, big)
        upd = cm < best_m
        best_m = jnp.where(upd, cm, best_m)
        bvmin = jnp.where(upd, s_lo, bvmin)
        bvmax = jnp.where(upd, s_hi, bvmax)
        bsv = jnp.where(upd, _roll(w1, o), bsv)
        bsv2 = jnp.where(upd, _roll(w2, o), bsv2)

    s1 = jnp.sum(bsv)
    s2 = jnp.sum(bsv2)
    s3 = jnp.sum(bsv * v)
    sx = jnp.sum(v)
    sx2 = jnp.sum(v * v)

    aw = w_ref[:, 0:1]
    cw = w_ref[:, 1:2] - aw
    g = g_ref[...].reshape(64, 1)
    bt = bt_ref[...].reshape(64, 1)
    m = f32(_B * _N * _K)
    mean = (aw * s1 + cw * (f32(_K) * sx)) / m
    ey2 = (aw * aw * s2 + 2.0 * aw * cw * s3 + cw * cw * (f32(_K) * sx2)) / m
    var = ey2 - mean * mean
    inv = jax.lax.rsqrt(var + 1e-5)
    amp = g * aw * inv
    cc = g * cw * inv
    d = bt - g * mean * inv

    x1s = []
    x2s = []
    for b in range(_B):
        sb = v[b:b + 1, :]
        vx = jnp.where(amp > 0, bvmax[b:b + 1, :], bvmin[b:b + 1, :])
        z = jnp.maximum(amp * vx + cc * sb + d, 0.0)
        x1s.append(jnp.max(z, axis=1, keepdims=True))
        x2s.append(jnp.sum(z, axis=1, keepdims=True) * f32(1.0 / _N))
    xh = jnp.concatenate([jnp.concatenate(x1s, axis=1),
                          jnp.concatenate(x2s, axis=1)], axis=0)
    out_ref[...] = jax.lax.dot_general(
        xh, lw_ref[...], (((0,), (1,)), ((), ())),
        preferred_element_type=jnp.float32)


def kernel(x, conv_w, bn_gamma, bn_beta, lin_w):
    return pl.pallas_call(
        _body,
        out_shape=jax.ShapeDtypeStruct((_B, _NC), jnp.float32),
    )(x, conv_w, bn_gamma, bn_beta, lin_w)


# speedup vs baseline: 1350.4542x; 1.2408x over previous
import jax
import jax.numpy as jnp
from jax.experimental import pallas as pl

_B = 4
_N = 4096
_K = 20
_NC = 40
_C = _N // 2


def _rl(v, s):
    if s % _C == 0:
        return v
    return jnp.roll(v, s, axis=1)


def _body(x_ref, w_ref, g_ref, bt_ref, lw_ref, out_ref):
    f32 = jnp.float32
    cl = jax.lax.broadcasted_iota(jnp.int32, (2 * _B, _C), 1)
    rp = jax.lax.broadcasted_iota(jnp.int32, (2 * _B, _C), 0) & 1
    fi = rp * _C + cl

    def down(v, o):
        a = _rl(v, o)
        return jnp.where(cl >= o, a, jnp.roll(a, 1, axis=0))

    def up(v, t):
        a = _rl(v, -t)
        return jnp.where(cl < _C - t, a, jnp.roll(a, -1, axis=0))

    rows = []
    for b in range(_B):
        rows.append(x_ref[b:b + 1, 0:_C])
        rows.append(x_ref[b:b + 1, _C:_N])
    v = jnp.concatenate(rows, axis=0)

    k = 2
    while k <= _N:
        j = k // 2
        while j >= 1:
            if j < _C:
                partner = jnp.where((cl & j) == 0, _rl(v, -j), _rl(v, j))
                low = (cl & j) == 0
            else:
                partner = jnp.where(rp == 0,
                                    jnp.roll(v, -1, axis=0),
                                    jnp.roll(v, 1, axis=0))
                low = rp == 0
            up_m = (fi & k) == 0
            keep_min = up_m == low
            v = jnp.where(keep_min, jnp.minimum(v, partner),
                          jnp.maximum(v, partner))
            j //= 2
        k *= 2

    vv = v * v
    a2 = v + up(v, 1)
    b2 = vv + up(vv, 1)
    a4 = a2 + up(a2, 2)
    b4 = b2 + up(b2, 2)
    a8 = a4 + up(a4, 4)
    b8 = b4 + up(b4, 4)
    a16 = a8 + up(a8, 8)
    b16 = b8 + up(b8, 8)
    w1 = a16 + up(a4, 16)
    w2 = b16 + up(b4, 16)

    big = f32(1e30)
    best_m = jnp.full((2 * _B, _C), big, f32)
    bvmin = jnp.zeros((2 * _B, _C), f32)
    bvmax = jnp.zeros((2 * _B, _C), f32)
    bsv = jnp.zeros((2 * _B, _C), f32)
    bsv2 = jnp.zeros((2 * _B, _C), f32)
    for o in range(_K):
        h = _K - 1 - o
        s_lo = down(v, o) if o else v
        s_hi = up(v, h) if h else v
        cm = jnp.maximum(v - s_lo, s_hi - v)
        valid = (fi >= o) & (fi <= (_N - _K) + o)
        cm = jnp.where(valid, cm, big)
        upd = cm < best_m
        best_m = jnp.where(upd, cm, best_m)
        bvmin = jnp.where(upd, s_lo, bvmin)
        bvmax = jnp.where(upd, s_hi, bvmax)
        wl = down(w1, o) if o else w1
        ql = down(w2, o) if o else w2
        bsv = jnp.where(upd, wl, bsv)
        bsv2 = jnp.where(upd, ql, bsv2)

    s1 = jnp.sum(bsv)
    s2 = jnp.sum(bsv2)
    s3 = jnp.sum(bsv * v)
    sx = jnp.sum(v)
    sx2 = jnp.sum(vv)

    aw = w_ref[:, 0:1]
    cw = w_ref[:, 1:2] - aw
    g = g_ref[...].reshape(64, 1)
    bt = bt_ref[...].reshape(64, 1)
    m = f32(_B * _N * _K)
    mean = (aw * s1 + cw * (f32(_K) * sx)) / m
    ey2 = (aw * aw * s2 + 2.0 * aw * cw * s3 + cw * cw * (f32(_K) * sx2)) / m
    var = ey2 - mean * mean
    inv = jax.lax.rsqrt(var + 1e-5)
    amp = g * aw * inv
    cc = g * cw * inv
    d = bt - g * mean * inv
    amp_p = jnp.maximum(amp, 0.0)
    amp_n = jnp.minimum(amp, 0.0)

    x1s = []
    x2s = []
    for b in range(_B):
        mx = None
        sm = None
        for r in (2 * b, 2 * b + 1):
            z = jnp.maximum(amp_p * bvmax[r:r + 1, :]
                            + amp_n * bvmin[r:r + 1, :]
                            + cc * v[r:r + 1, :] + d, 0.0)
            zm = jnp.max(z, axis=1, keepdims=True)
            zs = jnp.sum(z, axis=1, keepdims=True)
            mx = zm if mx is None else jnp.maximum(mx, zm)
            sm = zs if sm is None else sm + zs
        x1s.append(mx)
        x2s.append(sm * f32(1.0 / _N))
    xh = jnp.concatenate([jnp.concatenate(x1s, axis=1),
                          jnp.concatenate(x2s, axis=1)], axis=0)
    out_ref[...] = jax.lax.dot_general(
        xh, lw_ref[...], (((0,), (1,)), ((), ())),
        preferred_element_type=jnp.float32)


def kernel(x, conv_w, bn_gamma, bn_beta, lin_w):
    return pl.pallas_call(
        _body,
        out_shape=jax.ShapeDtypeStruct((_B, _NC), jnp.float32),
    )(x, conv_w, bn_gamma, bn_beta, lin_w)


# speedup vs baseline: 1418.1607x; 1.0501x over previous
import jax
import jax.numpy as jnp
from jax.experimental import pallas as pl

_B = 4
_N = 4096
_K = 20
_NC = 40
_C = _N // 2


def _rl(v, s):
    if s % _C == 0:
        return v
    return jnp.roll(v, s, axis=1)


def _body(x_ref, w_ref, g_ref, bt_ref, lw_ref, out_ref):
    f32 = jnp.float32
    cl = jax.lax.broadcasted_iota(jnp.int32, (2 * _B, _C), 1)
    rp = jax.lax.broadcasted_iota(jnp.int32, (2 * _B, _C), 0) & 1
    fi = rp * _C + cl

    def down(v, o, v_prev=None):
        if v_prev is None:
            v_prev = jnp.roll(v, 1, axis=0)
        return jnp.where(cl >= o, _rl(v, o), _rl(v_prev, o))

    def up(v, t, v_next=None):
        if v_next is None:
            v_next = jnp.roll(v, -1, axis=0)
        return jnp.where(cl < _C - t, _rl(v, -t), _rl(v_next, -t))

    rows = []
    for b in range(_B):
        rows.append(x_ref[b:b + 1, 0:_C])
        rows.append(x_ref[b:b + 1, _C:_N])
    v = jnp.concatenate(rows, axis=0)

    k = 2
    while k <= _N:
        j = k // 2
        while j >= 1:
            if j < _C:
                partner = jnp.where((cl & j) == 0, _rl(v, -j), _rl(v, j))
                low = (cl & j) == 0
            else:
                partner = jnp.where(rp == 0,
                                    jnp.roll(v, -1, axis=0),
                                    jnp.roll(v, 1, axis=0))
                low = rp == 0
            up_m = (fi & k) == 0
            keep_min = up_m == low
            v = jnp.where(keep_min, jnp.minimum(v, partner),
                          jnp.maximum(v, partner))
            j //= 2
        k *= 2

    vv = v * v
    a2 = v + up(v, 1)
    b2 = vv + up(vv, 1)
    a4 = a2 + up(a2, 2)
    b4 = b2 + up(b2, 2)
    a8 = a4 + up(a4, 4)
    b8 = b4 + up(b4, 4)
    a16 = a8 + up(a8, 8)
    b16 = b8 + up(b8, 8)
    w1 = a16 + up(a4, 16)
    w2 = b16 + up(b4, 16)

    big = f32(1e30)
    best_m = jnp.full((2 * _B, _C), big, f32)
    bvmin = jnp.zeros((2 * _B, _C), f32)
    bvmax = jnp.zeros((2 * _B, _C), f32)
    bsv = jnp.zeros((2 * _B, _C), f32)
    bsv2 = jnp.zeros((2 * _B, _C), f32)
    v_prev = jnp.roll(v, 1, axis=0)
    v_next = jnp.roll(v, -1, axis=0)
    w1_prev = jnp.roll(w1, 1, axis=0)
    w2_prev = jnp.roll(w2, 1, axis=0)
    for o in range(_K):
        h = _K - 1 - o
        s_lo = down(v, o, v_prev) if o else v
        s_hi = up(v, h, v_next) if h else v
        cm = jnp.maximum(v - s_lo, s_hi - v)
        valid = (fi >= o) & (fi <= (_N - _K) + o)
        cm = jnp.where(valid, cm, big)
        upd = cm < best_m
        best_m = jnp.where(upd, cm, best_m)
        bvmin = jnp.where(upd, s_lo, bvmin)
        bvmax = jnp.where(upd, s_hi, bvmax)
        wl = down(w1, o, w1_prev) if o else w1
        ql = down(w2, o, w2_prev) if o else w2
        bsv = jnp.where(upd, wl, bsv)
        bsv2 = jnp.where(upd, ql, bsv2)

    s1 = jnp.sum(bsv)
    s2 = jnp.sum(bsv2)
    s3 = jnp.sum(bsv * v)
    sx = jnp.sum(v)
    sx2 = jnp.sum(vv)

    aw = w_ref[:, 0:1]
    cw = w_ref[:, 1:2] - aw
    g = g_ref[...].reshape(64, 1)
    bt = bt_ref[...].reshape(64, 1)
    m = f32(_B * _N * _K)
    mean = (aw * s1 + cw * (f32(_K) * sx)) / m
    ey2 = (aw * aw * s2 + 2.0 * aw * cw * s3 + cw * cw * (f32(_K) * sx2)) / m
    var = ey2 - mean * mean
    inv = jax.lax.rsqrt(var + 1e-5)
    amp = g * aw * inv
    cc = g * cw * inv
    d = bt - g * mean * inv
    amp_p = jnp.maximum(amp, 0.0)
    amp_n = jnp.minimum(amp, 0.0)
    lhs = jnp.concatenate([amp_p, amp_n, cc, d], axis=1)

    ones = jnp.full((1, _C), 1.0, f32)
    x1s = []
    x2s = []
    for b in range(_B):
        mx = None
        sm = None
        for r in (2 * b, 2 * b + 1):
            rhs = jnp.concatenate([bvmax[r:r + 1, :], bvmin[r:r + 1, :],
                                   v[r:r + 1, :], ones], axis=0)
            z = jnp.maximum(jnp.dot(lhs, rhs,
                                    preferred_element_type=jnp.float32),
                            0.0)
            zm = jnp.max(z, axis=1, keepdims=True)
            zs = jnp.sum(z, axis=1, keepdims=True)
            mx = zm if mx is None else jnp.maximum(mx, zm)
            sm = zs if sm is None else sm + zs
        x1s.append(mx)
        x2s.append(sm * f32(1.0 / _N))
    xh = jnp.concatenate([jnp.concatenate(x1s, axis=1),
                          jnp.concatenate(x2s, axis=1)], axis=0)
    out_ref[...] = jax.lax.dot_general(
        xh, lw_ref[...], (((0,), (1,)), ((), ())),
        preferred_element_type=jnp.float32)


def kernel(x, conv_w, bn_gamma, bn_beta, lin_w):
    return pl.pallas_call(
        _body,
        out_shape=jax.ShapeDtypeStruct((_B, _NC), jnp.float32),
    )(x, conv_w, bn_gamma, bn_beta, lin_w)
